# Initial kernel scaffold; baseline (speedup 1.0000x reference)
import jax
import jax.numpy as jnp
from jax.experimental import pallas as pl


def kernel(context, W):
    raise NotImplementedError("write your pallas kernel here")


# speedup vs baseline: 4.8155x; 4.8155x over previous
import jax
import jax.numpy as jnp
from jax.experimental import pallas as pl
from jax.experimental.pallas import tpu as pltpu

_P = 16
_K = 64
_N = 2048
_D = 512
_PD = _P * _D
_BLK = 256
_CBLK = 256

_HI_KEY = 0x3FC00000
_LO_KEY = -0x3FC00001 - 1


def _v_body(ctx_ref, w_ref, v_ref):
    c = ctx_ref[...]
    w = w_ref[...]
    n2 = jax.lax.dot_general(
        c * c, w * w, (((1,), (1,)), ((), ())),
        preferred_element_type=jnp.float32,
        precision=jax.lax.Precision.HIGHEST)
    inv = 1.0 / jnp.maximum(jnp.sqrt(n2), 1e-12)
    for p in range(_P):
        v_ref[:, p * _D:(p + 1) * _D] = c * w[p][None, :] * inv[:, p][:, None]


def _att_body(vr_ref, vc_ref, out_ref):
    j = pl.program_id(1)
    nj = pl.num_programs(1)

    part = jax.lax.dot_general(
        vr_ref[...], vc_ref[...], (((1,), (1,)), ((), ())),
        preferred_element_type=jnp.float32,
        precision=jax.lax.Precision.DEFAULT)
    out_ref[:, pl.ds(j * _CBLK, _CBLK)] = part * (1.0 / _P)

    @pl.when(j == nj - 1)
    def _epilogue():
        att = out_ref[...]
        bits = jax.lax.bitcast_convert_type(att, jnp.int32)
        key = jnp.where(bits >= 0, bits, bits ^ 0x7FFFFFFF)

        lo0 = jnp.full((_BLK, 1), _LO_KEY, jnp.int32)
        hi0 = jnp.full((_BLK, 1), _HI_KEY, jnp.int32)

        def step(_, lh):
            lo, hi = lh
            mid = (lo + hi) >> 1
            cnt = jnp.sum((key > mid).astype(jnp.int32), axis=1,
                          keepdims=True)
            ge = cnt >= _K
            return jnp.where(ge, mid, lo), jnp.where(ge, hi, mid)

        lo, _ = jax.lax.fori_loop(0, 32, step, (lo0, hi0))
        out_ref[...] = jnp.where(key > lo, att, 0.0)


@jax.jit
def kernel(context, W):
    v = pl.pallas_call(
        _v_body,
        grid=(_N // _BLK,),
        in_specs=[
            pl.BlockSpec((_BLK, _D), lambda i: (i, 0)),
            pl.BlockSpec((_P, _D), lambda i: (0, 0)),
        ],
        out_specs=pl.BlockSpec((_BLK, _PD), lambda i: (i, 0)),
        out_shape=jax.ShapeDtypeStruct((_N, _PD), jnp.float32),
    )(context, W)

    return pl.pallas_call(
        _att_body,
        grid=(_N // _BLK, _N // _CBLK),
        in_specs=[
            pl.BlockSpec((_BLK, _PD), lambda i, j: (i, 0)),
            pl.BlockSpec((_CBLK, _PD), lambda i, j: (j, 0)),
        ],
        out_specs=pl.BlockSpec((_BLK, _N), lambda i, j: (i, 0)),
        out_shape=jax.ShapeDtypeStruct((_N, _N), jnp.float32),
    )(v, v)


# speedup vs baseline: 7.6909x; 1.5971x over previous
import jax
import jax.numpy as jnp
from jax.experimental import pallas as pl
from jax.experimental.pallas import tpu as pltpu

_P = 16
_K = 64
_N = 2048
_D = 512
_PD = _P * _D
_BLK = 512
_CBLK = 256

_HI_KEY = 0x3FC00000
_LO_KEY = -0x3FC00001 - 1


def _body(ctx_ref, w_ref, out_ref, vr_ref, vc_ref, inv_ref):
    i = pl.program_id(0)
    j = pl.program_id(1)
    nj = pl.num_programs(1)
    w = w_ref[...]

    @pl.when(jnp.logical_and(i == 0, j == 0))
    def _norms():
        c = ctx_ref[...]
        n2 = jax.lax.dot_general(
            c * c, w * w, (((1,), (1,)), ((), ())),
            preferred_element_type=jnp.float32,
            precision=jax.lax.Precision.HIGHEST)
        inv_ref[...] = 1.0 / jnp.maximum(jnp.sqrt(n2), 1e-12)

    @pl.when(j == 0)
    def _build_rows():
        c = ctx_ref[pl.ds(i * _BLK, _BLK), :]
        inv = inv_ref[pl.ds(i * _BLK, _BLK), :]
        for p in range(_P):
            vr_ref[:, p * _D:(p + 1) * _D] = (
                c * w[p][None, :] * inv[:, p][:, None])

    cj = ctx_ref[pl.ds(j * _CBLK, _CBLK), :]
    invj = inv_ref[pl.ds(j * _CBLK, _CBLK), :]
    for p in range(_P):
        vc_ref[:, p * _D:(p + 1) * _D] = (
            cj * w[p][None, :] * invj[:, p][:, None])

    part = jax.lax.dot_general(
        vr_ref[...], vc_ref[...], (((1,), (1,)), ((), ())),
        preferred_element_type=jnp.float32,
        precision=jax.lax.Precision.DEFAULT)
    out_ref[:, pl.ds(j * _CBLK, _CBLK)] = part * (1.0 / _P)

    @pl.when(j == nj - 1)
    def _epilogue():
        att = out_ref[...]
        bits = jax.lax.bitcast_convert_type(att, jnp.int32)
        key = jnp.where(bits >= 0, bits, bits ^ 0x7FFFFFFF)

        lo0 = jnp.full((_BLK, 1), _LO_KEY, jnp.int32)
        hi0 = jnp.full((_BLK, 1), _HI_KEY, jnp.int32)

        def step(_, lh):
            lo, hi = lh
            mid = (lo + hi) >> 1
            cnt = jnp.sum((key > mid).astype(jnp.int32), axis=1,
                          keepdims=True)
            ge = cnt >= _K
            return jnp.where(ge, mid, lo), jnp.where(ge, hi, mid)

        lo, _ = jax.lax.fori_loop(0, 32, step, (lo0, hi0))
        out_ref[...] = jnp.where(key > lo, att, 0.0)


@jax.jit
def kernel(context, W):
    return pl.pallas_call(
        _body,
        grid=(_N // _BLK, _N // _CBLK),
        in_specs=[
            pl.BlockSpec((_N, _D), lambda i, j: (0, 0)),
            pl.BlockSpec((_P, _D), lambda i, j: (0, 0)),
        ],
        out_specs=pl.BlockSpec((_BLK, _N), lambda i, j: (i, 0)),
        out_shape=jax.ShapeDtypeStruct((_N, _N), jnp.float32),
        scratch_shapes=[
            pltpu.VMEM((_BLK, _PD), jnp.float32),
            pltpu.VMEM((_CBLK, _PD), jnp.float32),
            pltpu.VMEM((_N, _P), jnp.float32),
        ],
    )(context, W)


# speedup vs baseline: 8.0918x; 1.0521x over previous
import jax
import jax.numpy as jnp
from jax.experimental import pallas as pl
from jax.experimental.pallas import tpu as pltpu

_P = 16
_K = 64
_N = 2048
_D = 512
_PD = _P * _D
_BLK = 512
_CBLK = 256

_HI_KEY = 0x3FC00000
_LO_KEY = -0x3FC00001 - 1


def _body(ctx_ref, w_ref, out_ref, vr_ref, vc_ref, inv_ref):
    i = pl.program_id(0)
    j = pl.program_id(1)
    nj = pl.num_programs(1)
    w = w_ref[...]

    @pl.when(jnp.logical_and(i == 0, j == 0))
    def _norms():
        c = ctx_ref[...]
        n2 = jax.lax.dot_general(
            c * c, w * w, (((1,), (1,)), ((), ())),
            preferred_element_type=jnp.float32,
            precision=jax.lax.Precision.HIGHEST)
        inv_ref[...] = 1.0 / jnp.maximum(jnp.sqrt(n2), 1e-12)

    @pl.when(j == 0)
    def _build_rows():
        c = ctx_ref[pl.ds(i * _BLK, _BLK), :]
        inv = inv_ref[pl.ds(i * _BLK, _BLK), :]
        for p in range(_P):
            vr_ref[:, p * _D:(p + 1) * _D] = (
                c * w[p][None, :] * inv[:, p][:, None])

    cj = ctx_ref[pl.ds(j * _CBLK, _CBLK), :]
    invj = inv_ref[pl.ds(j * _CBLK, _CBLK), :]
    for p in range(_P):
        vc_ref[:, p * _D:(p + 1) * _D] = (
            cj * w[p][None, :] * invj[:, p][:, None])

    part = jax.lax.dot_general(
        vr_ref[...], vc_ref[...], (((1,), (1,)), ((), ())),
        preferred_element_type=jnp.float32,
        precision=jax.lax.Precision.DEFAULT)
    out_ref[:, pl.ds(j * _CBLK, _CBLK)] = part * (1.0 / _P)

    @pl.when(j == nj - 1)
    def _epilogue():
        att = out_ref[...]
        ones = jnp.ones((1, _N), jnp.float32)

        def unmap(m):
            b = jnp.where(m >= 0, m, m ^ 0x7FFFFFFF)
            return jax.lax.bitcast_convert_type(b, jnp.float32)

        lo0 = jnp.full((_BLK, 1), _LO_KEY, jnp.int32)
        hi0 = jnp.full((_BLK, 1), _HI_KEY, jnp.int32)

        def step(_, lh):
            lo, hi = lh
            mid = (lo + hi) >> 1
            ind = jnp.where(att > unmap(mid), 1.0, 0.0)
            cnt = jax.lax.dot_general(
                ind, ones, (((1,), (1,)), ((), ())),
                preferred_element_type=jnp.float32)
            ge = cnt >= float(_K)
            return jnp.where(ge, mid, lo), jnp.where(ge, hi, mid)

        lo, _ = jax.lax.fori_loop(0, 32, step, (lo0, hi0))
        out_ref[...] = jnp.where(att > unmap(lo), att, 0.0)


@jax.jit
def kernel(context, W):
    return pl.pallas_call(
        _body,
        grid=(_N // _BLK, _N // _CBLK),
        in_specs=[
            pl.BlockSpec((_N, _D), lambda i, j: (0, 0)),
            pl.BlockSpec((_P, _D), lambda i, j: (0, 0)),
        ],
        out_specs=pl.BlockSpec((_BLK, _N), lambda i, j: (i, 0)),
        out_shape=jax.ShapeDtypeStruct((_N, _N), jnp.float32),
        scratch_shapes=[
            pltpu.VMEM((_BLK, _PD), jnp.float32),
            pltpu.VMEM((_CBLK, _PD), jnp.float32),
            pltpu.VMEM((_N, _P), jnp.float32),
        ],
    )(context, W)


# speedup vs baseline: 9.6414x; 1.1915x over previous
import jax
import jax.numpy as jnp
from jax.experimental import pallas as pl
from jax.experimental.pallas import tpu as pltpu

_P = 16
_K = 64
_N = 2048
_D = 512
_PD = _P * _D
_BLK = 512
_CBLK = 256

_HI_KEY = 0x3FC00000
_LO_KEY = -0x3FC00001 - 1


def _body(ctx_ref, w_ref, out_ref, vr_ref, vc_ref, inv_ref, mir_ref):
    i = pl.program_id(0)
    j = pl.program_id(1)
    nj = pl.num_programs(1)
    w = w_ref[...]

    @pl.when(jnp.logical_and(i == 0, j == 0))
    def _norms():
        c = ctx_ref[...]
        n2 = jax.lax.dot_general(
            c * c, w * w, (((1,), (1,)), ((), ())),
            preferred_element_type=jnp.float32,
            precision=jax.lax.Precision.HIGHEST)
        inv_ref[...] = 1.0 / jnp.maximum(jnp.sqrt(n2), 1e-12)

    @pl.when(j == 0)
    def _build_rows():
        c = ctx_ref[pl.ds(i * _BLK, _BLK), :]
        inv = inv_ref[pl.ds(i * _BLK, _BLK), :]
        for p in range(_P):
            vr_ref[:, p * _D:(p + 1) * _D] = (
                c * w[p][None, :] * inv[:, p][:, None])

    @pl.when(j >= 2 * i)
    def _upper():
        cj = ctx_ref[pl.ds(j * _CBLK, _CBLK), :]
        invj = inv_ref[pl.ds(j * _CBLK, _CBLK), :]
        for p in range(_P):
            vc_ref[:, p * _D:(p + 1) * _D] = (
                cj * w[p][None, :] * invj[:, p][:, None])

        part = jax.lax.dot_general(
            vr_ref[...], vc_ref[...], (((1,), (1,)), ((), ())),
            preferred_element_type=jnp.float32,
            precision=jax.lax.Precision.DEFAULT) * (1.0 / _P)
        out_ref[:, pl.ds(j * _CBLK, _CBLK)] = part

        @pl.when(j >= 2 * i + 2)
        def _stash_mirror():
            mir_ref[pl.ds(j * _CBLK, _CBLK), pl.ds(i * _BLK, _BLK)] = (
                jnp.transpose(part))

    @pl.when(j < 2 * i)
    def _copy_mirror():
        out_ref[:, pl.ds(j * _CBLK, _CBLK)] = (
            mir_ref[pl.ds(i * _BLK, _BLK), pl.ds(j * _CBLK, _CBLK)])

    @pl.when(j == nj - 1)
    def _epilogue():
        att = out_ref[...]
        ones = jnp.ones((1, _N), jnp.float32)

        def unmap(m):
            b = jnp.where(m >= 0, m, m ^ 0x7FFFFFFF)
            return jax.lax.bitcast_convert_type(b, jnp.float32)

        lo0 = jnp.full((_BLK, 1), _LO_KEY, jnp.int32)
        hi0 = jnp.full((_BLK, 1), _HI_KEY, jnp.int32)

        def step(_, lh):
            lo, hi = lh
            mid = (lo + hi) >> 1
            ind = jnp.where(att > unmap(mid), 1.0, 0.0)
            cnt = jax.lax.dot_general(
                ind, ones, (((1,), (1,)), ((), ())),
                preferred_element_type=jnp.float32)
            ge = cnt >= float(_K)
            return jnp.where(ge, mid, lo), jnp.where(ge, hi, mid)

        lo, _ = jax.lax.fori_loop(0, 32, step, (lo0, hi0))
        out_ref[...] = jnp.where(att > unmap(lo), att, 0.0)


@jax.jit
def kernel(context, W):
    return pl.pallas_call(
        _body,
        grid=(_N // _BLK, _N // _CBLK),
        in_specs=[
            pl.BlockSpec((_N, _D), lambda i, j: (0, 0)),
            pl.BlockSpec((_P, _D), lambda i, j: (0, 0)),
        ],
        out_specs=pl.BlockSpec((_BLK, _N), lambda i, j: (i, 0)),
        out_shape=jax.ShapeDtypeStruct((_N, _N), jnp.float32),
        scratch_shapes=[
            pltpu.VMEM((_BLK, _PD), jnp.float32),
            pltpu.VMEM((_CBLK, _PD), jnp.float32),
            pltpu.VMEM((_N, _P), jnp.float32),
            pltpu.VMEM((_N, _N - _BLK), jnp.float32),
        ],
    )(context, W)
